# baseline (speedup 1.0000x reference)
import jax
import jax.numpy as jnp
from jax import lax
from jax.experimental import pallas as pl
from jax.experimental.pallas import tpu as pltpu
from jax.experimental.pallas import tpu_sc as plsc

BATCH = 16384
N_FIELDS = 26
NC = 2
NS = 16
NW = NC * NS
B_PER_W = BATCH // NW
IDX_PER_W = B_PER_W * N_FIELDS
IDX_MINOR = 128
IDX_ROWS = IDX_PER_W // IDX_MINOR
ROWS_PER_J = B_PER_W // IDX_MINOR
GATHER_GROUP = 8
LANES = 16
TABLE_PAD = 1000448


def _sc_body(w_hbm, idx_hbm, out_hbm, idx_v, vals_v, out_v, sem):
    wid = lax.axis_index("s") * NC + lax.axis_index("c")
    base = wid * B_PER_W

    pltpu.sync_copy(idx_hbm.at[wid], idx_v)

    @pl.loop(0, IDX_ROWS, step=GATHER_GROUP)
    def _(r0):
        copies = [
            pltpu.async_copy(
                w_hbm.at[idx_v.at[r0 + i]], vals_v.at[r0 + i], sem
            )
            for i in range(GATHER_GROUP)
        ]
        for c in copies:
            c.wait()

    @pl.loop(0, ROWS_PER_J)
    def _(jr):
        @pl.loop(0, IDX_MINOR, step=LANES)
        def _(l0):
            acc = vals_v[jr, pl.ds(l0, LANES)]
            for f in range(1, N_FIELDS):
                acc = acc + vals_v[f * ROWS_PER_J + jr, pl.ds(l0, LANES)]
            out_v[pl.ds(jr * IDX_MINOR + l0, LANES)] = acc

    pltpu.sync_copy(out_v, out_hbm.at[pl.ds(base, B_PER_W)])


@jax.jit
def _sc_call(w_flat, idx_arranged):
    mesh = plsc.VectorSubcoreMesh(core_axis_name="c", subcore_axis_name="s")
    run = pl.kernel(
        _sc_body,
        out_type=jax.ShapeDtypeStruct((BATCH,), jnp.float32),
        mesh=mesh,
        scratch_types=[
            pltpu.VMEM((IDX_ROWS, IDX_MINOR), jnp.int32),
            pltpu.VMEM((IDX_ROWS, IDX_MINOR), jnp.float32),
            pltpu.VMEM((B_PER_W,), jnp.float32),
            pltpu.SemaphoreType.DMA,
        ],
    )
    return run(w_flat, idx_arranged)


def kernel(inputs, w):
    idx = inputs.astype(jnp.int32).T.reshape(N_FIELDS, NW, B_PER_W)
    idx = idx.transpose(1, 0, 2).reshape(NW, IDX_ROWS, IDX_MINOR)
    w_flat = jnp.pad(w.reshape(-1), (0, TABLE_PAD - w.shape[0]))
    out = _sc_call(w_flat, idx)
    return out.reshape(BATCH, 1)


# speedup vs baseline: 1.2920x; 1.2920x over previous
import dataclasses

import jax
import jax.numpy as jnp
from jax import lax
from jax.experimental import pallas as pl
from jax.experimental.pallas import tpu as pltpu
from jax.experimental.pallas import tpu_sc as plsc

BATCH = 16384
N_FIELDS = 26
NC = 2
NS = 16
NW = NC * NS
B_PER_W = BATCH // NW
IDX_PER_W = B_PER_W * N_FIELDS
IDX_MINOR = 128
ROWS_PER_F = B_PER_W // IDX_MINOR
LANES = 16
TABLE_PAD = 1000448


def _sc_body(w_hbm, idx_hbm, out_hbm, idx_v, vals_v, out_v, sem):
    wid = lax.axis_index("s") * NC + lax.axis_index("c")
    base = wid * B_PER_W

    pltpu.sync_copy(idx_hbm.at[:, wid], idx_v)

    @pl.loop(0, N_FIELDS, step=2)
    def _(f0):
        copies = [
            pltpu.async_copy(
                w_hbm.at[idx_v.at[f0 + i, q]],
                vals_v.at[pl.ds((f0 + i) * B_PER_W + q * IDX_MINOR, IDX_MINOR)],
                sem,
            )
            for i in range(2)
            for q in range(ROWS_PER_F)
        ]
        for c in copies:
            c.wait()

    @pl.loop(0, B_PER_W, step=LANES)
    def _(j0):
        acc = vals_v[pl.ds(j0, LANES)]
        for f in range(1, N_FIELDS):
            acc = acc + vals_v[pl.ds(f * B_PER_W + j0, LANES)]
        out_v[pl.ds(j0, LANES)] = acc

    pltpu.sync_copy(out_v, out_hbm.at[pl.ds(base, B_PER_W)])


@jax.jit
def _sc_call(w_flat, idx_t):
    mesh = plsc.VectorSubcoreMesh(core_axis_name="c", subcore_axis_name="s")
    cp = pltpu.CompilerParams()
    fields = pltpu.CompilerParams.__dataclass_fields__
    if "needs_layout_passes" in fields:
        cp = dataclasses.replace(cp, needs_layout_passes=False)
    if "use_tc_tiling_on_sc" in fields:
        cp = dataclasses.replace(cp, use_tc_tiling_on_sc=False)
    run = pl.kernel(
        _sc_body,
        compiler_params=cp,
        out_type=jax.ShapeDtypeStruct((BATCH,), jnp.float32),
        mesh=mesh,
        scratch_types=[
            pltpu.VMEM((N_FIELDS, ROWS_PER_F, IDX_MINOR), jnp.int32),
            pltpu.VMEM((IDX_PER_W,), jnp.float32),
            pltpu.VMEM((B_PER_W,), jnp.float32),
            pltpu.SemaphoreType.DMA,
        ],
    )
    return run(w_flat, idx_t)


def kernel(inputs, w):
    idx = inputs.astype(jnp.int32).T.reshape(N_FIELDS, NW, ROWS_PER_F, IDX_MINOR)
    split = (w.shape[0] // 1024) * 1024
    p1 = w[:split, :].reshape(-1)
    p2 = w[split:, :].reshape(-1)
    tail_zeros = jnp.zeros((TABLE_PAD - w.shape[0],), w.dtype)
    w_flat = jnp.concatenate([p1, p2, tail_zeros])
    out = _sc_call(w_flat, idx)
    return out.reshape(BATCH, 1)


# speedup vs baseline: 1.4742x; 1.1410x over previous
import dataclasses

import jax
import jax.numpy as jnp
from jax import lax
from jax.experimental import pallas as pl
from jax.experimental.pallas import tpu as pltpu
from jax.experimental.pallas import tpu_sc as plsc

BATCH = 16384
N_FIELDS = 26
NC = 2
NS = 16
NW = NC * NS
B_PER_W = BATCH // NW
IDX_PER_W = B_PER_W * N_FIELDS
IDX_MINOR = 128
ROWS_PER_F = B_PER_W // IDX_MINOR
LANES = 16
TABLE_PAD = 1000448


def _sc_body(w_hbm, idx_hbm, out_hbm, idx_v, vals_v, out_v, sem):
    wid = lax.axis_index("s") * NC + lax.axis_index("c")
    base = wid * B_PER_W

    pltpu.sync_copy(idx_hbm.at[:, wid], idx_v)

    @pl.loop(0, N_FIELDS)
    def _(f):
        for q in range(ROWS_PER_F):
            pltpu.async_copy(
                w_hbm.at[idx_v.at[f, q]],
                vals_v.at[pl.ds(f * B_PER_W + q * IDX_MINOR, IDX_MINOR)],
                sem,
            )
    pltpu.make_async_copy(w_hbm.at[pl.ds(0, IDX_PER_W)], vals_v, sem).wait()

    @pl.loop(0, B_PER_W, step=LANES)
    def _(j0):
        acc = vals_v[pl.ds(j0, LANES)]
        for f in range(1, N_FIELDS):
            acc = acc + vals_v[pl.ds(f * B_PER_W + j0, LANES)]
        out_v[pl.ds(j0, LANES)] = acc

    pltpu.sync_copy(out_v, out_hbm.at[pl.ds(base, B_PER_W)])


@jax.jit
def _sc_call(w_flat, idx_t):
    mesh = plsc.VectorSubcoreMesh(core_axis_name="c", subcore_axis_name="s")
    cp = pltpu.CompilerParams()
    fields = pltpu.CompilerParams.__dataclass_fields__
    if "needs_layout_passes" in fields:
        cp = dataclasses.replace(cp, needs_layout_passes=False)
    if "use_tc_tiling_on_sc" in fields:
        cp = dataclasses.replace(cp, use_tc_tiling_on_sc=False)
    run = pl.kernel(
        _sc_body,
        compiler_params=cp,
        out_type=jax.ShapeDtypeStruct((BATCH,), jnp.float32),
        mesh=mesh,
        scratch_types=[
            pltpu.VMEM((N_FIELDS, ROWS_PER_F, IDX_MINOR), jnp.int32),
            pltpu.VMEM((IDX_PER_W,), jnp.float32),
            pltpu.VMEM((B_PER_W,), jnp.float32),
            pltpu.SemaphoreType.DMA,
        ],
    )
    return run(w_flat, idx_t)


def kernel(inputs, w):
    idx = inputs.astype(jnp.int32).T.reshape(N_FIELDS, NW, ROWS_PER_F, IDX_MINOR)
    split = (w.shape[0] // 1024) * 1024
    p1 = w[:split, :].reshape(-1)
    p2 = w[split:, :].reshape(-1)
    tail_zeros = jnp.zeros((TABLE_PAD - w.shape[0],), w.dtype)
    w_flat = jnp.concatenate([p1, p2, tail_zeros])
    out = _sc_call(w_flat, idx)
    return out.reshape(BATCH, 1)
